# baseline (speedup 1.0000x reference)
import functools

import jax
import jax.numpy as jnp
from jax import lax
from jax.experimental import pallas as pl
from jax.experimental.pallas import tpu as pltpu
from jax.experimental.pallas import tpu_sc as plsc

N, C, H, W = 4, 96, 224, 224
P = H * W
R = 3 * N * P
NC, NS, L = 2, 16, 16
NW = NC * NS
RPW = R // NW
B = 64
CP = 128
SB = 14
NSB = RPW // (SB * B)
GROUPS = B // L

_SUBL = 392
_LANE = 128


def _corners(ix, iy, base, with_mask):
    x0f = jnp.floor(ix)
    y0f = jnp.floor(iy)
    x1f = x0f + 1.0
    y1f = y0f + 1.0
    wx1 = ix - x0f
    wx0 = 1.0 - wx1
    wy1 = iy - y0f
    wy0 = 1.0 - wy1

    def clampi(v, hi):
        return jnp.clip(v, 0.0, hi).astype(jnp.int32)

    xi0 = clampi(x0f, W - 1.0)
    xi1 = clampi(x1f, W - 1.0)
    yi0 = clampi(y0f, H - 1.0)
    yi1 = clampi(y1f, H - 1.0)

    w = [wy0 * wx0, wy0 * wx1, wy1 * wx0, wy1 * wx1]
    if with_mask:
        def m(yf, xf):
            return ((xf >= 0.0) & (xf <= W - 1.0)
                    & (yf >= 0.0) & (yf <= H - 1.0)).astype(jnp.float32)
        w[0] = w[0] * m(y0f, x0f)
        w[1] = w[1] * m(y0f, x1f)
        w[2] = w[2] * m(y1f, x0f)
        w[3] = w[3] * m(y1f, x1f)

    eq = (xi1 == xi0).astype(jnp.float32)
    ne = 1.0 - eq
    idx = (base + yi0 * W + xi0, base + yi1 * W + xi0)
    wgt = (w[0] + w[1] * eq, w[1] * ne, w[2] + w[3] * eq, w[3] * ne)
    return idx, wgt


def _reflect(c, span):
    cc = jnp.abs(c)
    extra = jnp.mod(cc, span)
    flips = jnp.floor(cc / span)
    return jnp.where(jnp.mod(flips, 2.0) == 0.0, extra, span - extra)


def _idx_weight_body(gx_ref, gy_ref, i0, i1, w0, w1, w2, w3):
    n = pl.program_id(0)
    base = n * P
    gx = gx_ref[0]
    gy = gy_ref[0]
    irefs = (i0, i1)
    wrefs = (w0, w1, w2, w3)

    ix = (gx + 1.0) * (0.5 * (W - 1.0))
    iy = (gy + 1.0) * (0.5 * (H - 1.0))
    idx, wgt = _corners(ix, iy, base, with_mask=True)
    for k in range(2):
        irefs[k][0, 0] = idx[k]
    for k in range(4):
        wrefs[k][0, 0] = wgt[k]

    ix = jnp.clip(((gx + 1.0) * W - 1.0) * 0.5, 0.0, W - 1.0)
    iy = jnp.clip(((gy + 1.0) * H - 1.0) * 0.5, 0.0, H - 1.0)
    idx, wgt = _corners(ix, iy, base, with_mask=False)
    for k in range(2):
        irefs[k][1, 0] = idx[k]
    for k in range(4):
        wrefs[k][1, 0] = wgt[k]

    ix = (gx + 1.0) * (0.5 * (W - 1.0))
    iy = (gy + 1.0) * (0.5 * (H - 1.0))
    ix = jnp.clip(_reflect(ix, W - 1.0), 0.0, W - 1.0)
    iy = jnp.clip(_reflect(iy, H - 1.0), 0.0, H - 1.0)
    idx, wgt = _corners(ix, iy, base, with_mask=False)
    for k in range(2):
        irefs[k][2, 0] = idx[k]
    for k in range(4):
        wrefs[k][2, 0] = wgt[k]


def _compute_idx_weights(gx, gy):
    ishape = jax.ShapeDtypeStruct((3, N, _SUBL, _LANE), jnp.int32)
    wshape = jax.ShapeDtypeStruct((3, N, _SUBL, _LANE), jnp.float32)
    in_spec = pl.BlockSpec((1, _SUBL, _LANE), lambda n: (n, 0, 0))
    out_spec = pl.BlockSpec((3, 1, _SUBL, _LANE), lambda n: (0, n, 0, 0))
    return pl.pallas_call(
        _idx_weight_body,
        grid=(N,),
        in_specs=[in_spec, in_spec],
        out_specs=[out_spec] * 6,
        out_shape=[ishape] * 2 + [wshape] * 4,
    )(gx, gy)


PAIRS = SB // 2
PPT = RPW // (2 * B)
PPI = P // (2 * B)


def _sc_body(table, i0, i1, w0, w1, w2, w3, out0, out1, out2,
             si0, si1, sw0, sw1, sw2, sw3,
             ga0, ga1, gb0, gb1,
             ov, semst, semg0, semg1, semo):
    wid = lax.axis_index("s") * NC + lax.axis_index("c")
    base0 = wid * RPW
    sidx = (si0, si1)
    swgt = (sw0, sw1, sw2, sw3)
    gbuf = ((ga0, ga1), (gb0, gb1))
    semg = (semg0, semg1)
    outs = (out0, out1, out2)

    def fire(cc, par):
        for k in range(2):
            pltpu.async_copy(
                table.at[sidx[k].at[pl.ds(cc * B, B)]], gbuf[par][k], semg[par])

    def drain(cc, par):
        for k in range(2):
            pltpu.make_async_copy(
                table.at[sidx[k].at[pl.ds(cc * B, B)]],
                gbuf[par][k], semg[par]).wait()

    def compute(cc, par, half):
        g0, g1 = gbuf[par]
        iota2 = 2 * lax.iota(jnp.int32, L)
        chanvs = [(cw * (2 * L) + iota2, cw * (2 * L) + 1 + iota2)
                  for cw in range(C // (2 * L))]
        himask = jnp.full((L,), -65536, jnp.int32)

        def group(gi, c2):
            woff = cc * B + gi * L
            wvecs = [swgt[k][pl.ds(woff, L)] for k in range(4)]
            for rl in range(L):
                r = gi * L + rl
                ws = [wvecs[k][rl] for k in range(4)]
                col = jnp.full((L,), half * B + r, jnp.int32)
                for cw in range(C // (2 * L)):
                    vs = [g0[r, pl.ds(cw * L, L)],
                          g0[r, pl.ds(48 + cw * L, L)],
                          g1[r, pl.ds(cw * L, L)],
                          g1[r, pl.ds(48 + cw * L, L)]]
                    acc_e = acc_o = None
                    for k in range(4):
                        ev = plsc.bitcast(vs[k] << 16, jnp.float32) * ws[k]
                        od = plsc.bitcast(vs[k] & himask, jnp.float32) * ws[k]
                        acc_e = ev if acc_e is None else acc_e + ev
                        acc_o = od if acc_o is None else acc_o + od
                    plsc.store_scatter(ov, [chanvs[cw][0], col], acc_e)
                    plsc.store_scatter(ov, [chanvs[cw][1], col], acc_o)
            return c2

        lax.fori_loop(0, GROUPS, group, 0)

    def out_wait():
        pltpu.make_async_copy(
            ov, out0.at[0, :, pl.ds(0, 2 * B)], semo).wait()

    def superblock(s, carry):
        rbase = s * SB * B
        ssl = pl.ds(base0 + rbase, SB * B)
        stcps = [pltpu.async_copy(src.at[ssl], dst, semst)
                 for src, dst in zip((i0, i1), sidx)]
        stcps += [pltpu.async_copy(src.at[ssl], dst, semst)
                  for src, dst in zip((w0, w1, w2, w3), swgt)]
        for cp in stcps:
            cp.wait()

        fire(0, 0)

        def pair(j, c2):
            cc0 = 2 * j
            gp_local = s * PAIRS + j
            gp = wid * PPT + gp_local
            m = gp // (PPI * N)
            rem = gp - m * (PPI * N)
            n = rem // PPI
            p0 = (rem - n * PPI) * (2 * B)

            fire(cc0 + 1, 1)
            drain(cc0, 0)

            @pl.when(gp_local > 0)
            def _():
                out_wait()

            compute(cc0, 0, half=0)

            @pl.when(j < PAIRS - 1)
            def _():
                fire(cc0 + 2, 0)

            drain(cc0 + 1, 1)
            compute(cc0 + 1, 1, half=1)

            for mo in range(3):
                @pl.when(m == mo)
                def _(mo=mo):
                    pltpu.async_copy(
                        ov, outs[mo].at[n, :, pl.ds(p0, 2 * B)], semo)
            return c2

        lax.fori_loop(0, PAIRS, pair, 0)
        return carry

    lax.fori_loop(0, NSB, superblock, 0)
    out_wait()


@functools.cache
def _sc_gather():
    oshape = jax.ShapeDtypeStruct((N, C, P), jnp.float32)
    return pl.kernel(
        _sc_body,
        out_type=[oshape] * 3,
        mesh=plsc.VectorSubcoreMesh(
            core_axis_name="c", subcore_axis_name="s",
            num_cores=NC, num_subcores=NS),
        compiler_params=pltpu.CompilerParams(needs_layout_passes=False),
        scratch_types=[pltpu.VMEM((SB * B,), jnp.int32)] * 2
        + [pltpu.VMEM((SB * B,), jnp.float32)] * 4
        + [pltpu.VMEM((B, CP), jnp.int32)] * 4
        + [pltpu.VMEM((C, 2 * B), jnp.float32),
           pltpu.SemaphoreType.DMA,
           pltpu.SemaphoreType.DMA,
           pltpu.SemaphoreType.DMA,
           pltpu.SemaphoreType.DMA],
    )


def kernel(x, T):
    xb = x.transpose(0, 2, 3, 1).astype(jnp.bfloat16)
    xr = jnp.concatenate([xb[:, :, 1:], xb[:, :, -1:]], axis=2)
    pad = jnp.zeros((N, H, W, 2 * CP - 2 * C), jnp.bfloat16)
    pair = jnp.concatenate([xb, xr, pad], axis=3)
    x_flat = jax.lax.bitcast_convert_type(
        pair.reshape(N * P, CP, 2), jnp.int32)
    gx = T[..., 0].reshape(N, _SUBL, _LANE)
    gy = T[..., 1].reshape(N, _SUBL, _LANE)
    i0, i1, w0, w1, w2, w3 = _compute_idx_weights(gx, gy)
    flat = lambda a: a.reshape(R)
    y0, y1, y2 = _sc_gather()(x_flat,
                              flat(i0), flat(i1),
                              flat(w0), flat(w1), flat(w2), flat(w3))
    return (y0.reshape(N, C, H, W),
            y1.reshape(N, C, H, W),
            y2.reshape(N, C, H, W))


# speedup vs baseline: 1.1480x; 1.1480x over previous
import functools

import jax
import jax.numpy as jnp
from jax import lax
from jax.experimental import pallas as pl
from jax.experimental.pallas import tpu as pltpu
from jax.experimental.pallas import tpu_sc as plsc

N, C, H, W = 4, 96, 224, 224
P = H * W
R = 3 * N * P
NC, NS, L = 2, 16, 16
NW = NC * NS
RPW = R // NW
B = 64
CP = 128
SB = 14
NSB = RPW // (SB * B)
GROUPS = B // L

_SUBL = 392
_LANE = 128


def _corners(ix, iy, base, with_mask):
    x0f = jnp.floor(ix)
    y0f = jnp.floor(iy)
    x1f = x0f + 1.0
    y1f = y0f + 1.0
    wx1 = ix - x0f
    wx0 = 1.0 - wx1
    wy1 = iy - y0f
    wy0 = 1.0 - wy1

    def clampi(v, hi):
        return jnp.clip(v, 0.0, hi).astype(jnp.int32)

    xi0 = clampi(x0f, W - 1.0)
    xi1 = clampi(x1f, W - 1.0)
    yi0 = clampi(y0f, H - 1.0)
    yi1 = clampi(y1f, H - 1.0)

    w = [wy0 * wx0, wy0 * wx1, wy1 * wx0, wy1 * wx1]
    if with_mask:
        def m(yf, xf):
            return ((xf >= 0.0) & (xf <= W - 1.0)
                    & (yf >= 0.0) & (yf <= H - 1.0)).astype(jnp.float32)
        w[0] = w[0] * m(y0f, x0f)
        w[1] = w[1] * m(y0f, x1f)
        w[2] = w[2] * m(y1f, x0f)
        w[3] = w[3] * m(y1f, x1f)

    eq = (xi1 == xi0).astype(jnp.float32)
    ne = 1.0 - eq
    idx = (base + yi0 * W + xi0, base + yi1 * W + xi0)
    wgt = (w[0] + w[1] * eq, w[1] * ne, w[2] + w[3] * eq, w[3] * ne)
    return idx, wgt


def _reflect(c, span):
    cc = jnp.abs(c)
    extra = jnp.mod(cc, span)
    flips = jnp.floor(cc / span)
    return jnp.where(jnp.mod(flips, 2.0) == 0.0, extra, span - extra)


def _idx_weight_body(gx_ref, gy_ref, i0, i1, w0, w1, w2, w3):
    n = pl.program_id(0)
    base = n * P
    gx = gx_ref[0]
    gy = gy_ref[0]
    irefs = (i0, i1)
    wrefs = (w0, w1, w2, w3)

    ix = (gx + 1.0) * (0.5 * (W - 1.0))
    iy = (gy + 1.0) * (0.5 * (H - 1.0))
    idx, wgt = _corners(ix, iy, base, with_mask=True)
    for k in range(2):
        irefs[k][0, 0] = idx[k]
    for k in range(4):
        wrefs[k][0, 0] = wgt[k]

    ix = jnp.clip(((gx + 1.0) * W - 1.0) * 0.5, 0.0, W - 1.0)
    iy = jnp.clip(((gy + 1.0) * H - 1.0) * 0.5, 0.0, H - 1.0)
    idx, wgt = _corners(ix, iy, base, with_mask=False)
    for k in range(2):
        irefs[k][1, 0] = idx[k]
    for k in range(4):
        wrefs[k][1, 0] = wgt[k]

    ix = (gx + 1.0) * (0.5 * (W - 1.0))
    iy = (gy + 1.0) * (0.5 * (H - 1.0))
    ix = jnp.clip(_reflect(ix, W - 1.0), 0.0, W - 1.0)
    iy = jnp.clip(_reflect(iy, H - 1.0), 0.0, H - 1.0)
    idx, wgt = _corners(ix, iy, base, with_mask=False)
    for k in range(2):
        irefs[k][2, 0] = idx[k]
    for k in range(4):
        wrefs[k][2, 0] = wgt[k]


def _compute_idx_weights(gx, gy):
    ishape = jax.ShapeDtypeStruct((3, N, _SUBL, _LANE), jnp.int32)
    wshape = jax.ShapeDtypeStruct((3, N, _SUBL, _LANE), jnp.float32)
    in_spec = pl.BlockSpec((1, _SUBL, _LANE), lambda n: (n, 0, 0))
    out_spec = pl.BlockSpec((3, 1, _SUBL, _LANE), lambda n: (0, n, 0, 0))
    return pl.pallas_call(
        _idx_weight_body,
        grid=(N,),
        in_specs=[in_spec, in_spec],
        out_specs=[out_spec] * 6,
        out_shape=[ishape] * 2 + [wshape] * 4,
    )(gx, gy)


PAIRS = SB // 2
PPT = RPW // (2 * B)
PPI = P // (2 * B)


def _sc_body(table, i0, i1, w0, w1, w2, w3, out0, out1, out2,
             si0, si1, sw0, sw1, sw2, sw3,
             ga0, ga1, gb0, gb1,
             ov, semst, semg0, semg1, semo0, semo1):
    wid = lax.axis_index("s") * NC + lax.axis_index("c")
    base0 = wid * RPW
    sidx = (si0, si1)
    swgt = (sw0, sw1, sw2, sw3)
    gbuf = ((ga0, ga1), (gb0, gb1))
    semg = (semg0, semg1)
    semo = (semo0, semo1)
    outs = (out0, out1, out2)

    def fire(cc, par):
        for k in range(2):
            pltpu.async_copy(
                table.at[sidx[k].at[pl.ds(cc * B, B)]], gbuf[par][k], semg[par])

    def drain(cc, par):
        for k in range(2):
            pltpu.make_async_copy(
                table.at[sidx[k].at[pl.ds(cc * B, B)]],
                gbuf[par][k], semg[par]).wait()

    def compute(cc, par, colbase):
        g0, g1 = gbuf[par]
        iota2 = 2 * lax.iota(jnp.int32, L)
        chanvs = [(cw * (2 * L) + iota2, cw * (2 * L) + 1 + iota2)
                  for cw in range(C // (2 * L))]
        himask = jnp.full((L,), -65536, jnp.int32)

        def group(gi, c2):
            woff = cc * B + gi * L
            wvecs = [swgt[k][pl.ds(woff, L)] for k in range(4)]
            for rl in range(L):
                r = gi * L + rl
                ws = [wvecs[k][rl] for k in range(4)]
                col = jnp.full((L,), colbase + r, jnp.int32)
                for cw in range(C // (2 * L)):
                    vs = [g0[r, pl.ds(cw * L, L)],
                          g0[r, pl.ds(48 + cw * L, L)],
                          g1[r, pl.ds(cw * L, L)],
                          g1[r, pl.ds(48 + cw * L, L)]]
                    acc_e = acc_o = None
                    for k in range(4):
                        ev = plsc.bitcast(vs[k] << 16, jnp.float32) * ws[k]
                        od = plsc.bitcast(vs[k] & himask, jnp.float32) * ws[k]
                        acc_e = ev if acc_e is None else acc_e + ev
                        acc_o = od if acc_o is None else acc_o + od
                    plsc.store_scatter(ov, [chanvs[cw][0], col], acc_e)
                    plsc.store_scatter(ov, [chanvs[cw][1], col], acc_o)
            return c2

        lax.fori_loop(0, GROUPS, group, 0)

    def out_wait(q):
        pltpu.make_async_copy(
            ov.at[:, pl.ds(q * 2 * B, 2 * B)],
            out0.at[0, :, pl.ds(0, 2 * B)], semo[q]).wait()

    def superblock(s, carry):
        rbase = s * SB * B
        ssl = pl.ds(base0 + rbase, SB * B)
        stcps = [pltpu.async_copy(src.at[ssl], dst, semst)
                 for src, dst in zip((i0, i1), sidx)]
        stcps += [pltpu.async_copy(src.at[ssl], dst, semst)
                  for src, dst in zip((w0, w1, w2, w3), swgt)]
        for cp in stcps:
            cp.wait()

        fire(0, 0)

        def pair(j, c2):
            cc0 = 2 * j
            gp_local = s * PAIRS + j
            gp = wid * PPT + gp_local
            m = gp // (PPI * N)
            rem = gp - m * (PPI * N)
            n = rem // PPI
            p0 = (rem - n * PPI) * (2 * B)

            po = gp_local & 1

            fire(cc0 + 1, 1)
            drain(cc0, 0)

            for q in range(2):
                @pl.when((gp_local >= 2) & (po == q))
                def _(q=q):
                    out_wait(q)

            compute(cc0, 0, colbase=po * (2 * B))

            @pl.when(j < PAIRS - 1)
            def _():
                fire(cc0 + 2, 0)

            drain(cc0 + 1, 1)
            compute(cc0 + 1, 1, colbase=po * (2 * B) + B)

            for q in range(2):
                for mo in range(3):
                    @pl.when((po == q) & (m == mo))
                    def _(q=q, mo=mo):
                        pltpu.async_copy(
                            ov.at[:, pl.ds(q * 2 * B, 2 * B)],
                            outs[mo].at[n, :, pl.ds(p0, 2 * B)], semo[q])
            return c2

        lax.fori_loop(0, PAIRS, pair, 0)
        return carry

    lax.fori_loop(0, NSB, superblock, 0)
    out_wait(0)
    out_wait(1)


@functools.cache
def _sc_gather():
    oshape = jax.ShapeDtypeStruct((N, C, P), jnp.float32)
    return pl.kernel(
        _sc_body,
        out_type=[oshape] * 3,
        mesh=plsc.VectorSubcoreMesh(
            core_axis_name="c", subcore_axis_name="s",
            num_cores=NC, num_subcores=NS),
        compiler_params=pltpu.CompilerParams(needs_layout_passes=False),
        scratch_types=[pltpu.VMEM((SB * B,), jnp.int32)] * 2
        + [pltpu.VMEM((SB * B,), jnp.float32)] * 4
        + [pltpu.VMEM((B, CP), jnp.int32)] * 4
        + [pltpu.VMEM((C, 4 * B), jnp.float32),
           pltpu.SemaphoreType.DMA,
           pltpu.SemaphoreType.DMA,
           pltpu.SemaphoreType.DMA,
           pltpu.SemaphoreType.DMA,
           pltpu.SemaphoreType.DMA],
    )


def kernel(x, T):
    xb = x.transpose(0, 2, 3, 1).astype(jnp.bfloat16).reshape(N * P, C)
    xr = jnp.concatenate([xb[1:], xb[-1:]], axis=0)
    pad = jnp.zeros((N * P, 2 * CP - 2 * C), jnp.bfloat16)
    pair = jnp.concatenate([xb, xr, pad], axis=1)
    x_flat = jax.lax.bitcast_convert_type(
        pair.reshape(N * P, CP, 2), jnp.int32)
    gx = T[..., 0].reshape(N, _SUBL, _LANE)
    gy = T[..., 1].reshape(N, _SUBL, _LANE)
    i0, i1, w0, w1, w2, w3 = _compute_idx_weights(gx, gy)
    flat = lambda a: a.reshape(R)
    y0, y1, y2 = _sc_gather()(x_flat,
                              flat(i0), flat(i1),
                              flat(w0), flat(w1), flat(w2), flat(w3))
    return (y0.reshape(N, C, H, W),
            y1.reshape(N, C, H, W),
            y2.reshape(N, C, H, W))


# speedup vs baseline: 1.1480x; 1.0000x over previous
import functools

import jax
import jax.numpy as jnp
from jax import lax
from jax.experimental import pallas as pl
from jax.experimental.pallas import tpu as pltpu
from jax.experimental.pallas import tpu_sc as plsc

N, C, H, W = 4, 96, 224, 224
P = H * W
R = 3 * N * P
NC, NS, L = 2, 16, 16
NW = NC * NS
RPW = R // NW
B = 64
CP = 128
SB = 14
NSB = RPW // (SB * B)
GROUPS = B // L

_SUBL = 392
_LANE = 128


def _corners(ix, iy, base, with_mask):
    x0f = jnp.floor(ix)
    y0f = jnp.floor(iy)
    x1f = x0f + 1.0
    y1f = y0f + 1.0
    wx1 = ix - x0f
    wx0 = 1.0 - wx1
    wy1 = iy - y0f
    wy0 = 1.0 - wy1

    def clampi(v, hi):
        return jnp.clip(v, 0.0, hi).astype(jnp.int32)

    xi0 = clampi(x0f, W - 1.0)
    xi1 = clampi(x1f, W - 1.0)
    yi0 = clampi(y0f, H - 1.0)
    yi1 = clampi(y1f, H - 1.0)

    w = [wy0 * wx0, wy0 * wx1, wy1 * wx0, wy1 * wx1]
    if with_mask:
        def m(yf, xf):
            return ((xf >= 0.0) & (xf <= W - 1.0)
                    & (yf >= 0.0) & (yf <= H - 1.0)).astype(jnp.float32)
        w[0] = w[0] * m(y0f, x0f)
        w[1] = w[1] * m(y0f, x1f)
        w[2] = w[2] * m(y1f, x0f)
        w[3] = w[3] * m(y1f, x1f)

    eq = (xi1 == xi0).astype(jnp.float32)
    ne = 1.0 - eq
    idx = (base + yi0 * W + xi0, base + yi1 * W + xi0)
    wgt = (w[0] + w[1] * eq, w[1] * ne, w[2] + w[3] * eq, w[3] * ne)
    return idx, wgt


def _reflect(c, span):
    cc = jnp.abs(c)
    extra = jnp.mod(cc, span)
    flips = jnp.floor(cc / span)
    return jnp.where(jnp.mod(flips, 2.0) == 0.0, extra, span - extra)


def _idx_weight_body(gx_ref, gy_ref, i0, i1, w0, w1, w2, w3):
    n = pl.program_id(0)
    base = n * P
    gx = gx_ref[0]
    gy = gy_ref[0]
    irefs = (i0, i1)
    wrefs = (w0, w1, w2, w3)

    ix = (gx + 1.0) * (0.5 * (W - 1.0))
    iy = (gy + 1.0) * (0.5 * (H - 1.0))
    idx, wgt = _corners(ix, iy, base, with_mask=True)
    for k in range(2):
        irefs[k][0, 0] = idx[k]
    for k in range(4):
        wrefs[k][0, 0] = wgt[k]

    ix = jnp.clip(((gx + 1.0) * W - 1.0) * 0.5, 0.0, W - 1.0)
    iy = jnp.clip(((gy + 1.0) * H - 1.0) * 0.5, 0.0, H - 1.0)
    idx, wgt = _corners(ix, iy, base, with_mask=False)
    for k in range(2):
        irefs[k][1, 0] = idx[k]
    for k in range(4):
        wrefs[k][1, 0] = wgt[k]

    ix = (gx + 1.0) * (0.5 * (W - 1.0))
    iy = (gy + 1.0) * (0.5 * (H - 1.0))
    ix = jnp.clip(_reflect(ix, W - 1.0), 0.0, W - 1.0)
    iy = jnp.clip(_reflect(iy, H - 1.0), 0.0, H - 1.0)
    idx, wgt = _corners(ix, iy, base, with_mask=False)
    for k in range(2):
        irefs[k][2, 0] = idx[k]
    for k in range(4):
        wrefs[k][2, 0] = wgt[k]


def _compute_idx_weights(gx, gy):
    ishape = jax.ShapeDtypeStruct((3, N, _SUBL, _LANE), jnp.int32)
    wshape = jax.ShapeDtypeStruct((3, N, _SUBL, _LANE), jnp.float32)
    in_spec = pl.BlockSpec((1, _SUBL, _LANE), lambda n: (n, 0, 0))
    out_spec = pl.BlockSpec((3, 1, _SUBL, _LANE), lambda n: (0, n, 0, 0))
    return pl.pallas_call(
        _idx_weight_body,
        grid=(N,),
        in_specs=[in_spec, in_spec],
        out_specs=[out_spec] * 6,
        out_shape=[ishape] * 2 + [wshape] * 4,
    )(gx, gy)


PAIRS = SB // 2
PPT = RPW // (2 * B)
PPI = P // (2 * B)


def _sc_body(table, i0, i1, w0, w1, w2, w3, out0, out1, out2,
             si0, si1, sw0, sw1, sw2, sw3,
             ga0, ga1, gb0, gb1,
             ov, semst, semg0, semg1, semo0, semo1):
    wid = lax.axis_index("s") * NC + lax.axis_index("c")
    base0 = wid * RPW
    sidx = (si0, si1)
    swgt = (sw0, sw1, sw2, sw3)
    gbuf = ((ga0, ga1), (gb0, gb1))
    semg = (semg0, semg1)
    semo = (semo0, semo1)
    outs = (out0, out1, out2)

    def fire(cc, par):
        for k in range(2):
            pltpu.async_copy(
                table.at[sidx[k].at[pl.ds(cc * B, B)]], gbuf[par][k], semg[par])

    def drain(cc, par):
        for k in range(2):
            pltpu.make_async_copy(
                table.at[sidx[k].at[pl.ds(cc * B, B)]],
                gbuf[par][k], semg[par]).wait()

    def compute(cc, par, colbase):
        g0, g1 = gbuf[par]
        iota2 = 2 * lax.iota(jnp.int32, L)
        chanvs = [(cw * (2 * L) + iota2, cw * (2 * L) + 1 + iota2)
                  for cw in range(C // (2 * L))]
        himask = jnp.full((L,), -65536, jnp.int32)

        def group(gi, c2):
            woff = cc * B + gi * L
            wvecs = [swgt[k][pl.ds(woff, L)] for k in range(4)]
            for rl in range(L):
                r = gi * L + rl
                ws = [wvecs[k][rl] for k in range(4)]
                col = jnp.full((L,), colbase + r, jnp.int32)
                for cw in range(C // (2 * L)):
                    vs = [g0[r, pl.ds(cw * L, L)],
                          g0[r, pl.ds(48 + cw * L, L)],
                          g1[r, pl.ds(cw * L, L)],
                          g1[r, pl.ds(48 + cw * L, L)]]
                    acc_e = acc_o = None
                    for k in range(4):
                        ev = plsc.bitcast(vs[k] << 16, jnp.float32) * ws[k]
                        od = plsc.bitcast(vs[k] & himask, jnp.float32) * ws[k]
                        acc_e = ev if acc_e is None else acc_e + ev
                        acc_o = od if acc_o is None else acc_o + od
                    plsc.store_scatter(ov, [chanvs[cw][0], col], acc_e)
                    plsc.store_scatter(ov, [chanvs[cw][1], col], acc_o)
            return c2

        lax.fori_loop(0, GROUPS, group, 0)

    def out_wait(q):
        pltpu.make_async_copy(
            ov.at[:, pl.ds(q * 2 * B, 2 * B)],
            out0.at[0, :, pl.ds(0, 2 * B)], semo[q]).wait()

    def superblock(s, carry):
        rbase = s * SB * B
        ssl = pl.ds(base0 + rbase, SB * B)
        stcps = [pltpu.async_copy(src.at[ssl], dst, semst)
                 for src, dst in zip((i0, i1), sidx)]
        stcps += [pltpu.async_copy(src.at[ssl], dst, semst)
                  for src, dst in zip((w0, w1, w2, w3), swgt)]
        for cp in stcps:
            cp.wait()

        fire(0, 0)

        def pair(j, c2):
            cc0 = 2 * j
            gp_local = s * PAIRS + j
            gp = wid * PPT + gp_local
            m = gp // (PPI * N)
            rem = gp - m * (PPI * N)
            n = rem // PPI
            p0 = (rem - n * PPI) * (2 * B)

            po = gp_local & 1

            fire(cc0 + 1, 1)
            drain(cc0, 0)

            for q in range(2):
                @pl.when((gp_local >= 2) & (po == q))
                def _(q=q):
                    out_wait(q)

            compute(cc0, 0, colbase=po * (2 * B))

            @pl.when(j < PAIRS - 1)
            def _():
                fire(cc0 + 2, 0)

            drain(cc0 + 1, 1)
            compute(cc0 + 1, 1, colbase=po * (2 * B) + B)

            for q in range(2):
                for mo in range(3):
                    @pl.when((po == q) & (m == mo))
                    def _(q=q, mo=mo):
                        pltpu.async_copy(
                            ov.at[:, pl.ds(q * 2 * B, 2 * B)],
                            outs[mo].at[n, :, pl.ds(p0, 2 * B)], semo[q])
            return c2

        lax.fori_loop(0, PAIRS, pair, 0)
        return carry

    lax.fori_loop(0, NSB, superblock, 0)
    out_wait(0)
    out_wait(1)


@functools.cache
def _sc_gather():
    oshape = jax.ShapeDtypeStruct((N, C, P), jnp.float32)
    return pl.kernel(
        _sc_body,
        out_type=[oshape] * 3,
        mesh=plsc.VectorSubcoreMesh(
            core_axis_name="c", subcore_axis_name="s",
            num_cores=NC, num_subcores=NS),
        compiler_params=pltpu.CompilerParams(needs_layout_passes=False),
        scratch_types=[pltpu.VMEM((SB * B,), jnp.int32)] * 2
        + [pltpu.VMEM((SB * B,), jnp.float32)] * 4
        + [pltpu.VMEM((B, CP), jnp.int32)] * 4
        + [pltpu.VMEM((C, 4 * B), jnp.float32),
           pltpu.SemaphoreType.DMA,
           pltpu.SemaphoreType.DMA,
           pltpu.SemaphoreType.DMA,
           pltpu.SemaphoreType.DMA,
           pltpu.SemaphoreType.DMA],
    )


def kernel(x, T):
    xb = jax.lax.optimization_barrier(
        x.transpose(0, 2, 3, 1).astype(jnp.bfloat16).reshape(N * P, C))
    xr = jnp.concatenate([xb[1:], xb[-1:]], axis=0)
    pad = jnp.zeros((N * P, 2 * CP - 2 * C), jnp.bfloat16)
    pair = jnp.concatenate([xb, xr, pad], axis=1)
    x_flat = jax.lax.bitcast_convert_type(
        pair.reshape(N * P, CP, 2), jnp.int32)
    gx = T[..., 0].reshape(N, _SUBL, _LANE)
    gy = T[..., 1].reshape(N, _SUBL, _LANE)
    i0, i1, w0, w1, w2, w3 = _compute_idx_weights(gx, gy)
    flat = lambda a: a.reshape(R)
    y0, y1, y2 = _sc_gather()(x_flat,
                              flat(i0), flat(i1),
                              flat(w0), flat(w1), flat(w2), flat(w3))
    return (y0.reshape(N, C, H, W),
            y1.reshape(N, C, H, W),
            y2.reshape(N, C, H, W))


# speedup vs baseline: 1.2955x; 1.1285x over previous
import functools

import jax
import jax.numpy as jnp
from jax import lax
from jax.experimental import pallas as pl
from jax.experimental.pallas import tpu as pltpu
from jax.experimental.pallas import tpu_sc as plsc

N, C, H, W = 4, 96, 224, 224
P = H * W
R = 3 * N * P
NC, NS, L = 2, 16, 16
NW = NC * NS
RPW = R // NW
B = 64
CP = 128
SB = 14
NSB = RPW // (SB * B)
GROUPS = B // L

_SUBL = 392
_LANE = 128


def _corners(ix, iy, base, with_mask):
    x0f = jnp.floor(ix)
    y0f = jnp.floor(iy)
    x1f = x0f + 1.0
    y1f = y0f + 1.0
    wx1 = ix - x0f
    wx0 = 1.0 - wx1
    wy1 = iy - y0f
    wy0 = 1.0 - wy1

    def clampi(v, hi):
        return jnp.clip(v, 0.0, hi).astype(jnp.int32)

    xi0 = clampi(x0f, W - 1.0)
    xi1 = clampi(x1f, W - 1.0)
    yi0 = clampi(y0f, H - 1.0)
    yi1 = clampi(y1f, H - 1.0)

    w = [wy0 * wx0, wy0 * wx1, wy1 * wx0, wy1 * wx1]
    if with_mask:
        def m(yf, xf):
            return ((xf >= 0.0) & (xf <= W - 1.0)
                    & (yf >= 0.0) & (yf <= H - 1.0)).astype(jnp.float32)
        w[0] = w[0] * m(y0f, x0f)
        w[1] = w[1] * m(y0f, x1f)
        w[2] = w[2] * m(y1f, x0f)
        w[3] = w[3] * m(y1f, x1f)

    eq = (xi1 == xi0).astype(jnp.float32)
    ne = 1.0 - eq
    idx = (base + yi0 * W + xi0, base + yi1 * W + xi0)
    wgt = (w[0] + w[1] * eq, w[1] * ne, w[2] + w[3] * eq, w[3] * ne)
    return idx, wgt


def _reflect(c, span):
    cc = jnp.abs(c)
    extra = jnp.mod(cc, span)
    flips = jnp.floor(cc / span)
    return jnp.where(jnp.mod(flips, 2.0) == 0.0, extra, span - extra)


def _idx_weight_body(gx_ref, gy_ref, i0, i1, w0, w1, w2, w3):
    n = pl.program_id(0)
    base = n * P
    gx = gx_ref[0]
    gy = gy_ref[0]
    irefs = (i0, i1)
    wrefs = (w0, w1, w2, w3)

    ix = (gx + 1.0) * (0.5 * (W - 1.0))
    iy = (gy + 1.0) * (0.5 * (H - 1.0))
    idx, wgt = _corners(ix, iy, base, with_mask=True)
    for k in range(2):
        irefs[k][0, 0] = idx[k]
    for k in range(4):
        wrefs[k][0, 0] = wgt[k]

    ix = jnp.clip(((gx + 1.0) * W - 1.0) * 0.5, 0.0, W - 1.0)
    iy = jnp.clip(((gy + 1.0) * H - 1.0) * 0.5, 0.0, H - 1.0)
    idx, wgt = _corners(ix, iy, base, with_mask=False)
    for k in range(2):
        irefs[k][1, 0] = idx[k]
    for k in range(4):
        wrefs[k][1, 0] = wgt[k]

    ix = (gx + 1.0) * (0.5 * (W - 1.0))
    iy = (gy + 1.0) * (0.5 * (H - 1.0))
    ix = jnp.clip(_reflect(ix, W - 1.0), 0.0, W - 1.0)
    iy = jnp.clip(_reflect(iy, H - 1.0), 0.0, H - 1.0)
    idx, wgt = _corners(ix, iy, base, with_mask=False)
    for k in range(2):
        irefs[k][2, 0] = idx[k]
    for k in range(4):
        wrefs[k][2, 0] = wgt[k]


def _compute_idx_weights(gx, gy):
    ishape = jax.ShapeDtypeStruct((3, N, _SUBL, _LANE), jnp.int32)
    wshape = jax.ShapeDtypeStruct((3, N, _SUBL, _LANE), jnp.float32)
    in_spec = pl.BlockSpec((1, _SUBL, _LANE), lambda n: (n, 0, 0))
    out_spec = pl.BlockSpec((3, 1, _SUBL, _LANE), lambda n: (0, n, 0, 0))
    return pl.pallas_call(
        _idx_weight_body,
        grid=(N,),
        in_specs=[in_spec, in_spec],
        out_specs=[out_spec] * 6,
        out_shape=[ishape] * 2 + [wshape] * 4,
    )(gx, gy)


PAIRS = SB // 2
PPT = RPW // (2 * B)
PPI = P // (2 * B)


def _sc_body(table, i0, i1, w0, w1, w2, w3, out0, out1, out2,
             si0, si1, sw0, sw1, sw2, sw3,
             ga0, ga1, gb0, gb1,
             ov, semst, semg0, semg1, semo0, semo1):
    wid = lax.axis_index("s") * NC + lax.axis_index("c")
    base0 = wid * RPW
    sidx = (si0, si1)
    swgt = (sw0, sw1, sw2, sw3)
    gbuf = ((ga0, ga1), (gb0, gb1))
    semg = (semg0, semg1)
    semo = (semo0, semo1)
    outs = (out0, out1, out2)

    def fire(cc, par):
        for k in range(2):
            pltpu.async_copy(
                table.at[sidx[k].at[pl.ds(cc * B, B)]], gbuf[par][k], semg[par])

    def drain(cc, par):
        for k in range(2):
            pltpu.make_async_copy(
                table.at[sidx[k].at[pl.ds(cc * B, B)]],
                gbuf[par][k], semg[par]).wait()

    def compute(cc, par, colbase):
        g0, g1 = gbuf[par]
        iota = lax.iota(jnp.int32, L)
        chanvs = [(cw * L + iota, C // 2 + cw * L + iota)
                  for cw in range(C // (2 * L))]
        himask = jnp.full((L,), -65536, jnp.int32)

        def group(gi, c2):
            woff = cc * B + gi * L
            wvecs = [swgt[k][pl.ds(woff, L)] for k in range(4)]
            for rl in range(L):
                r = gi * L + rl
                ws = [wvecs[k][rl] for k in range(4)]
                col = jnp.full((L,), colbase + r, jnp.int32)
                for cw in range(C // (2 * L)):
                    vs = [g0[r, pl.ds(cw * L, L)],
                          g0[r, pl.ds(48 + cw * L, L)],
                          g1[r, pl.ds(cw * L, L)],
                          g1[r, pl.ds(48 + cw * L, L)]]
                    acc_e = acc_o = None
                    for k in range(4):
                        ev = plsc.bitcast(vs[k] << 16, jnp.float32) * ws[k]
                        od = plsc.bitcast(vs[k] & himask, jnp.float32) * ws[k]
                        acc_e = ev if acc_e is None else acc_e + ev
                        acc_o = od if acc_o is None else acc_o + od
                    plsc.store_scatter(ov, [chanvs[cw][0], col], acc_e)
                    plsc.store_scatter(ov, [chanvs[cw][1], col], acc_o)
            return c2

        lax.fori_loop(0, GROUPS, group, 0)

    def out_wait(q):
        pltpu.make_async_copy(
            ov.at[:, pl.ds(q * 2 * B, 2 * B)],
            out0.at[0, :, pl.ds(0, 2 * B)], semo[q]).wait()

    def superblock(s, carry):
        rbase = s * SB * B
        ssl = pl.ds(base0 + rbase, SB * B)
        stcps = [pltpu.async_copy(src.at[ssl], dst, semst)
                 for src, dst in zip((i0, i1), sidx)]
        stcps += [pltpu.async_copy(src.at[ssl], dst, semst)
                  for src, dst in zip((w0, w1, w2, w3), swgt)]
        for cp in stcps:
            cp.wait()

        fire(0, 0)

        def pair(j, c2):
            cc0 = 2 * j
            gp_local = s * PAIRS + j
            gp = wid * PPT + gp_local
            m = gp // (PPI * N)
            rem = gp - m * (PPI * N)
            n = rem // PPI
            p0 = (rem - n * PPI) * (2 * B)

            po = gp_local & 1

            fire(cc0 + 1, 1)
            drain(cc0, 0)

            for q in range(2):
                @pl.when((gp_local >= 2) & (po == q))
                def _(q=q):
                    out_wait(q)

            compute(cc0, 0, colbase=po * (2 * B))

            @pl.when(j < PAIRS - 1)
            def _():
                fire(cc0 + 2, 0)

            drain(cc0 + 1, 1)
            compute(cc0 + 1, 1, colbase=po * (2 * B) + B)

            for q in range(2):
                for mo in range(3):
                    @pl.when((po == q) & (m == mo))
                    def _(q=q, mo=mo):
                        pltpu.async_copy(
                            ov.at[:, pl.ds(q * 2 * B, 2 * B)],
                            outs[mo].at[n, :, pl.ds(p0, 2 * B)], semo[q])
            return c2

        lax.fori_loop(0, PAIRS, pair, 0)
        return carry

    lax.fori_loop(0, NSB, superblock, 0)
    out_wait(0)
    out_wait(1)


@functools.cache
def _sc_gather():
    oshape = jax.ShapeDtypeStruct((N, C, P), jnp.float32)
    return pl.kernel(
        _sc_body,
        out_type=[oshape] * 3,
        mesh=plsc.VectorSubcoreMesh(
            core_axis_name="c", subcore_axis_name="s",
            num_cores=NC, num_subcores=NS),
        compiler_params=pltpu.CompilerParams(needs_layout_passes=False),
        scratch_types=[pltpu.VMEM((SB * B,), jnp.int32)] * 2
        + [pltpu.VMEM((SB * B,), jnp.float32)] * 4
        + [pltpu.VMEM((B, CP), jnp.int32)] * 4
        + [pltpu.VMEM((C, 4 * B), jnp.float32),
           pltpu.SemaphoreType.DMA,
           pltpu.SemaphoreType.DMA,
           pltpu.SemaphoreType.DMA,
           pltpu.SemaphoreType.DMA,
           pltpu.SemaphoreType.DMA],
    )


def kernel(x, T):
    u = jax.lax.bitcast_convert_type(
        x.transpose(0, 2, 3, 1), jnp.uint32).reshape(N * P, C)
    v = (u + jnp.uint32(0x7FFF) + ((u >> 16) & jnp.uint32(1))) >> 16
    w48 = v[:, : C // 2] | (v[:, C // 2:] << 16)
    wr = jnp.concatenate([w48[1:], w48[-1:]], axis=0)
    pad = jnp.zeros((N * P, CP - C), jnp.uint32)
    x_flat = jax.lax.bitcast_convert_type(
        jnp.concatenate([w48, wr, pad], axis=1), jnp.int32)
    gx = T[..., 0].reshape(N, _SUBL, _LANE)
    gy = T[..., 1].reshape(N, _SUBL, _LANE)
    i0, i1, w0, w1, w2, w3 = _compute_idx_weights(gx, gy)
    flat = lambda a: a.reshape(R)
    y0, y1, y2 = _sc_gather()(x_flat,
                              flat(i0), flat(i1),
                              flat(w0), flat(w1), flat(w2), flat(w3))
    return (y0.reshape(N, C, H, W),
            y1.reshape(N, C, H, W),
            y2.reshape(N, C, H, W))
